# Initial kernel scaffold; baseline (speedup 1.0000x reference)
import jax
import jax.numpy as jnp
from jax.experimental import pallas as pl


def kernel(x, edge_index, edge_attr, W1, root1, b1, W2, root2, b2, W3, root3, b3, W4, root4, b4, lw1, lb1, lw2, lb2):
    raise NotImplementedError("write your pallas kernel here")


# speedup vs baseline: 7.2832x; 7.2832x over previous
import functools

import jax
import jax.numpy as jnp
from jax import lax
from jax.experimental import pallas as pl
from jax.experimental.pallas import tpu as pltpu
from jax.experimental.pallas import tpu_sc as plsc

KS = 5
S = 4
K = 25
N = 10000
E = 160000
NCLS = 6890

NC = 2
NS = 16
NW = NC * NS
EPAD = 163840
EW = EPAD // NW
B = 128
CH = EW // B
NSH = 10240


def _prep_body(p0, p1, src, dst, bas, flat, dsto):
    pid = pl.program_id(0)
    r = lax.broadcasted_iota(jnp.int32, (8, 1024), 0)
    c = lax.broadcasted_iota(jnp.int32, (8, 1024), 1)
    eid = (pid * 8 + r) * 1024 + c
    valid = eid < E

    v0 = p0[...] * (KS - 1.0)
    lo0 = jnp.floor(v0)
    f0 = v0 - lo0
    lo0i = lo0.astype(jnp.int32)
    hi0i = jnp.minimum(lo0i + 1, KS - 1)
    v1 = p1[...] * (KS - 1.0)
    lo1 = jnp.floor(v1)
    f1 = v1 - lo1
    lo1i = lo1.astype(jnp.int32)
    hi1i = jnp.minimum(lo1i + 1, KS - 1)
    sv = src[...]
    for s in range(S):
        b = (f0 if (s & 1) else (1.0 - f0)) * (f1 if ((s >> 1) & 1) else (1.0 - f1))
        wi = (hi0i if (s & 1) else lo0i) + KS * (hi1i if ((s >> 1) & 1) else lo1i)
        bas[s] = jnp.where(valid, b, 0.0)
        flat[s] = jnp.where(valid, sv * K + wi, 0)
    dsto[...] = dst[...]


_prep = pl.pallas_call(
    _prep_body,
    grid=(20,),
    in_specs=[pl.BlockSpec((8, 1024), lambda i: (i, 0)) for _ in range(4)],
    out_specs=[
        pl.BlockSpec((S, 8, 1024), lambda i: (0, i, 0)),
        pl.BlockSpec((S, 8, 1024), lambda i: (0, i, 0)),
        pl.BlockSpec((8, 1024), lambda i: (i, 0)),
    ],
    out_shape=[
        jax.ShapeDtypeStruct((S, 160, 1024), jnp.float32),
        jax.ShapeDtypeStruct((S, 160, 1024), jnp.int32),
        jax.ShapeDtypeStruct((160, 1024), jnp.int32),
    ],
)


def _transform(x, wmat, in_ch, ko):
    def body(x_ref, w_ref, o_ref):
        if in_ch == 1:
            o_ref[...] = x_ref[...] * w_ref[...]
        else:
            o_ref[...] = jnp.dot(x_ref[...], w_ref[...],
                                 preferred_element_type=jnp.float32)

    return pl.pallas_call(
        body,
        grid=(25,),
        in_specs=[
            pl.BlockSpec((400, in_ch), lambda i: (i, 0)),
            pl.BlockSpec((in_ch, ko), lambda i: (0, 0)),
        ],
        out_specs=pl.BlockSpec((400, ko), lambda i: (i, 0)),
        out_shape=jax.ShapeDtypeStruct((N, ko), jnp.float32),
    )(x, wmat)


def _combine(aggp, x, root, bias, in_ch, out_ch):
    def body(a_ref, x_ref, r_ref, b_ref, o_ref):
        if in_ch == 1:
            rt = x_ref[...] * r_ref[...]
        else:
            rt = jnp.dot(x_ref[...], r_ref[...],
                         preferred_element_type=jnp.float32)
        h = a_ref[0] + a_ref[1] + rt + b_ref[...]
        o_ref[...] = jnp.where(h > 0, h, jnp.exp(jnp.minimum(h, 0.0)) - 1.0)

    return pl.pallas_call(
        body,
        grid=(25,),
        in_specs=[
            pl.BlockSpec((NC, 400, out_ch), lambda i: (0, i, 0)),
            pl.BlockSpec((400, in_ch), lambda i: (i, 0)),
            pl.BlockSpec((in_ch, out_ch), lambda i: (0, 0)),
            pl.BlockSpec((1, out_ch), lambda i: (0, 0)),
        ],
        out_specs=pl.BlockSpec((400, out_ch), lambda i: (i, 0)),
        out_shape=jax.ShapeDtypeStruct((N, out_ch), jnp.float32),
    )(aggp, x, root, bias.reshape(1, -1))


def _head(h, lw1, lb1, lw2, lb2):
    RB = 200

    def body(h_ref, w1, b1, w2, b2, o_ref):
        h1 = jnp.dot(h_ref[...], w1[...], preferred_element_type=jnp.float32)
        h1 = h1 + b1[...]
        h1 = jnp.where(h1 > 0, h1, jnp.exp(jnp.minimum(h1, 0.0)) - 1.0)
        lg = jnp.dot(h1, w2[...], preferred_element_type=jnp.float32)
        lg = lg + b2[...]
        m = jnp.max(lg, axis=1, keepdims=True)
        lse = jnp.log(jnp.sum(jnp.exp(lg - m), axis=1, keepdims=True))
        o_ref[...] = lg - m - lse

    return pl.pallas_call(
        body,
        grid=(N // RB,),
        in_specs=[
            pl.BlockSpec((RB, 64), lambda i: (i, 0)),
            pl.BlockSpec((64, 256), lambda i: (0, 0)),
            pl.BlockSpec((1, 256), lambda i: (0, 0)),
            pl.BlockSpec((256, NCLS), lambda i: (0, 0)),
            pl.BlockSpec((1, NCLS), lambda i: (0, 0)),
        ],
        out_specs=pl.BlockSpec((RB, NCLS), lambda i: (i, 0)),
        out_shape=jax.ShapeDtypeStruct((N, NCLS), jnp.float32),
    )(h, lw1, lb1.reshape(1, -1), lw2, lb2.reshape(1, -1))


def _make_sc(out_ch):
    F = out_ch // 16
    mesh = plsc.VectorSubcoreMesh(core_axis_name="c", subcore_axis_name="s")

    @functools.partial(
        pl.kernel,
        out_type=jax.ShapeDtypeStruct((NC, NSH, out_ch), jnp.float32),
        mesh=mesh,
        compiler_params=pltpu.CompilerParams(use_tc_tiling_on_sc=False),
        scratch_types=[
            pltpu.VMEM((3, S, B), jnp.int32),
            pltpu.VMEM((3, S, B), jnp.float32),
            pltpu.VMEM((3, 1, B), jnp.int32),
            pltpu.VMEM((2, S, B, out_ch), jnp.float32),
            pltpu.VMEM((2, B, out_ch), jnp.float32),
            pltpu.VMEM_SHARED((NSH, out_ch), jnp.float32),
            pltpu.SemaphoreType.DMA,
            pltpu.SemaphoreType.DMA,
            pltpu.SemaphoreType.DMA,
        ],
    )
    def sc_fn(xk, basT, flatT, dste, out, idx_v, bas_v, dst_v, rows_v,
              msg_v, agg, sem_m, sem_g, sem_s):
        c = lax.axis_index("c")
        sid = lax.axis_index("s")
        wid = c * NS + sid

        z16 = jnp.zeros((16,), jnp.float32)

        def zrow(r, carry):
            for j in range(F):
                msg_v[0, r, pl.ds(j * 16, 16)] = z16
            return carry

        lax.fori_loop(0, B, zrow, 0)
        for i in range(NSH // NS // B):
            pltpu.sync_copy(msg_v.at[0],
                            agg.at[pl.ds(sid * (NSH // NS) + i * B, B)])
        plsc.subcore_barrier()

        def fire_meta(ci):
            m = lax.rem(ci, 3)
            base = wid * EW + ci * B
            pltpu.async_copy(flatT.at[:, pl.ds(base, B)], idx_v.at[m], sem_m)
            pltpu.async_copy(basT.at[:, pl.ds(base, B)], bas_v.at[m], sem_m)
            pltpu.async_copy(dste.at[pl.ds(base, B)], dst_v.at[m, 0], sem_m)

        def wait_meta():
            pltpu.make_async_copy(flatT.at[:, pl.ds(0, B)], idx_v.at[0],
                                  sem_m).wait()
            pltpu.make_async_copy(basT.at[:, pl.ds(0, B)], bas_v.at[0],
                                  sem_m).wait()
            pltpu.make_async_copy(dste.at[pl.ds(0, B)], dst_v.at[0, 0],
                                  sem_m).wait()

        def fire_gathers(ci):
            m = lax.rem(ci, 3)
            p = lax.rem(ci, 2)
            for si in range(S):
                pltpu.async_copy(xk.at[idx_v.at[m, si]], rows_v.at[p, si],
                                 sem_g)

        def wait_gathers():
            for si in range(S):
                pltpu.make_async_copy(xk.at[idx_v.at[0, si]],
                                      rows_v.at[0, si], sem_g).wait()

        fire_meta(jnp.int32(0))
        fire_meta(jnp.int32(1))
        wait_meta()
        fire_gathers(jnp.int32(0))

        def wait_scatter():
            pltpu.make_async_copy(msg_v.at[0], agg.at[dst_v.at[0, 0]],
                                  sem_s).wait()

        def chunk(ci, carry):
            m = lax.rem(ci, 3)
            p = lax.rem(ci, 2)
            wait_gathers()

            @pl.when(ci + 1 < CH)
            def _():
                wait_meta()
                fire_gathers(ci + 1)

            @pl.when(ci >= 1)
            def _():
                wait_scatter()

            @pl.when(ci + 2 < CH)
            def _():
                fire_meta(ci + 2)

            def group(g, gcarry):
                gbase = pl.multiple_of(g * 16, 16)
                bvecs = [bas_v[m, si, pl.ds(gbase, 16)] for si in range(S)]
                for i in range(16):
                    e = gbase + i
                    bb = [bvecs[si][i] for si in range(S)]
                    for j in range(F):
                        acc = bb[0] * rows_v[p, 0, e, pl.ds(j * 16, 16)]
                        for si in range(1, S):
                            acc = acc + bb[si] * rows_v[p, si, e,
                                                        pl.ds(j * 16, 16)]
                        msg_v[p, e, pl.ds(j * 16, 16)] = acc
                return gcarry

            lax.fori_loop(0, B // 16, group, 0)
            pltpu.async_copy(msg_v.at[p], agg.at[dst_v.at[m, 0]], sem_s,
                             add=True)
            return carry

        lax.fori_loop(0, CH, chunk, 0)
        wait_scatter()
        plsc.subcore_barrier()
        rt = NSH // NS
        pltpu.sync_copy(agg.at[pl.ds(sid * rt, rt)],
                        out.at[c, pl.ds(sid * rt, rt)])

    return sc_fn


_sc32 = _make_sc(32)
_sc64 = _make_sc(64)


def kernel(x, edge_index, edge_attr, W1, root1, b1, W2, root2, b2,
           W3, root3, b3, W4, root4, b4, lw1, lb1, lw2, lb2):
    ei = edge_index.astype(jnp.int32)
    pad = EPAD - E
    p0 = jnp.pad(edge_attr[:, 0], (0, pad)).reshape(160, 1024)
    p1 = jnp.pad(edge_attr[:, 1], (0, pad)).reshape(160, 1024)
    srcp = jnp.pad(ei[0], (0, pad)).reshape(160, 1024)
    dstp = jnp.pad(ei[1], (0, pad)).reshape(160, 1024)

    bas4, flat4, dst4 = _prep(p0, p1, srcp, dstp)
    basT = bas4.reshape(S, EPAD)
    flatT = flat4.reshape(S, EPAD)
    dste = dst4.reshape(EPAD)

    def layer(h, W, root, bias, in_ch, out_ch, sc_fn):
        wmat = W.transpose(1, 0, 2).reshape(in_ch, K * out_ch)
        xk = _transform(h, wmat, in_ch, K * out_ch).reshape(N * K, out_ch)
        aggp = sc_fn(xk, basT, flatT, dste)
        return _combine(aggp, h, root, bias, in_ch, out_ch)

    h = layer(x, W1, root1, b1, 1, 32, _sc32)
    h = layer(h, W2, root2, b2, 32, 64, _sc64)
    h = layer(h, W3, root3, b3, 64, 64, _sc64)
    h = layer(h, W4, root4, b4, 64, 64, _sc64)
    return _head(h, lw1, lb1, lw2, lb2)


# speedup vs baseline: 7.3753x; 1.0126x over previous
import functools

import jax
import jax.numpy as jnp
from jax import lax
from jax.experimental import pallas as pl
from jax.experimental.pallas import tpu as pltpu
from jax.experimental.pallas import tpu_sc as plsc

KS = 5
S = 4
K = 25
N = 10000
E = 160000
NCLS = 6890

NC = 2
NS = 16
NW = NC * NS
EPAD = 163840
EW = EPAD // NW
B = 128
CH = EW // B
NSH = 10240


def _prep_body(p0, p1, src, dst, bas, flat, dsto):
    pid = pl.program_id(0)
    r = lax.broadcasted_iota(jnp.int32, (8, 1024), 0)
    c = lax.broadcasted_iota(jnp.int32, (8, 1024), 1)
    eid = (pid * 8 + r) * 1024 + c
    valid = eid < E

    v0 = p0[...] * (KS - 1.0)
    lo0 = jnp.floor(v0)
    f0 = v0 - lo0
    lo0i = lo0.astype(jnp.int32)
    hi0i = jnp.minimum(lo0i + 1, KS - 1)
    v1 = p1[...] * (KS - 1.0)
    lo1 = jnp.floor(v1)
    f1 = v1 - lo1
    lo1i = lo1.astype(jnp.int32)
    hi1i = jnp.minimum(lo1i + 1, KS - 1)
    sv = src[...]
    for s in range(S):
        b = (f0 if (s & 1) else (1.0 - f0)) * (f1 if ((s >> 1) & 1) else (1.0 - f1))
        wi = (hi0i if (s & 1) else lo0i) + KS * (hi1i if ((s >> 1) & 1) else lo1i)
        bas[s] = jnp.where(valid, b, 0.0)
        flat[s] = jnp.where(valid, sv * K + wi, 0)
    dsto[...] = dst[...]


_prep = pl.pallas_call(
    _prep_body,
    grid=(20,),
    in_specs=[pl.BlockSpec((8, 1024), lambda i: (i, 0)) for _ in range(4)],
    out_specs=[
        pl.BlockSpec((S, 8, 1024), lambda i: (0, i, 0)),
        pl.BlockSpec((S, 8, 1024), lambda i: (0, i, 0)),
        pl.BlockSpec((8, 1024), lambda i: (i, 0)),
    ],
    out_shape=[
        jax.ShapeDtypeStruct((S, 160, 1024), jnp.float32),
        jax.ShapeDtypeStruct((S, 160, 1024), jnp.int32),
        jax.ShapeDtypeStruct((160, 1024), jnp.int32),
    ],
)


def _transform(x, wmat, in_ch, ko):
    def body(x_ref, w_ref, o_ref):
        if in_ch == 1:
            o_ref[...] = x_ref[...] * w_ref[...]
        else:
            o_ref[...] = jnp.dot(x_ref[...], w_ref[...],
                                 preferred_element_type=jnp.float32)

    return pl.pallas_call(
        body,
        grid=(25,),
        in_specs=[
            pl.BlockSpec((400, in_ch), lambda i: (i, 0)),
            pl.BlockSpec((in_ch, ko), lambda i: (0, 0)),
        ],
        out_specs=pl.BlockSpec((400, ko), lambda i: (i, 0)),
        out_shape=jax.ShapeDtypeStruct((N, ko), jnp.float32),
    )(x, wmat)


def _combine(aggp, x, root, bias, in_ch, out_ch):
    def body(a_ref, x_ref, r_ref, b_ref, o_ref):
        if in_ch == 1:
            rt = x_ref[...] * r_ref[...]
        else:
            rt = jnp.dot(x_ref[...], r_ref[...],
                         preferred_element_type=jnp.float32)
        h = a_ref[0] + a_ref[1] + rt + b_ref[...]
        o_ref[...] = jnp.where(h > 0, h, jnp.exp(jnp.minimum(h, 0.0)) - 1.0)

    return pl.pallas_call(
        body,
        grid=(25,),
        in_specs=[
            pl.BlockSpec((NC, 400, out_ch), lambda i: (0, i, 0)),
            pl.BlockSpec((400, in_ch), lambda i: (i, 0)),
            pl.BlockSpec((in_ch, out_ch), lambda i: (0, 0)),
            pl.BlockSpec((1, out_ch), lambda i: (0, 0)),
        ],
        out_specs=pl.BlockSpec((400, out_ch), lambda i: (i, 0)),
        out_shape=jax.ShapeDtypeStruct((N, out_ch), jnp.float32),
    )(aggp, x, root, bias.reshape(1, -1))


def _head(h, lw1, lb1, lw2, lb2):
    RB = 200

    def body(h_ref, w1, b1, w2, b2, o_ref):
        h1 = jnp.dot(h_ref[...], w1[...], preferred_element_type=jnp.float32)
        h1 = h1 + b1[...]
        h1 = jnp.where(h1 > 0, h1, jnp.exp(jnp.minimum(h1, 0.0)) - 1.0)
        lg = jnp.dot(h1, w2[...], preferred_element_type=jnp.float32)
        lg = lg + b2[...]
        m = jnp.max(lg, axis=1, keepdims=True)
        lse = jnp.log(jnp.sum(jnp.exp(lg - m), axis=1, keepdims=True))
        o_ref[...] = lg - m - lse

    return pl.pallas_call(
        body,
        grid=(N // RB,),
        in_specs=[
            pl.BlockSpec((RB, 64), lambda i: (i, 0)),
            pl.BlockSpec((64, 256), lambda i: (0, 0)),
            pl.BlockSpec((1, 256), lambda i: (0, 0)),
            pl.BlockSpec((256, NCLS), lambda i: (0, 0)),
            pl.BlockSpec((1, NCLS), lambda i: (0, 0)),
        ],
        out_specs=pl.BlockSpec((RB, NCLS), lambda i: (i, 0)),
        out_shape=jax.ShapeDtypeStruct((N, NCLS), jnp.float32),
    )(h, lw1, lb1.reshape(1, -1), lw2, lb2.reshape(1, -1))


def _make_sc(out_ch):
    F = out_ch // 16
    mesh = plsc.VectorSubcoreMesh(core_axis_name="c", subcore_axis_name="s")

    @functools.partial(
        pl.kernel,
        out_type=jax.ShapeDtypeStruct((NC, NSH, out_ch), jnp.float32),
        mesh=mesh,
        compiler_params=pltpu.CompilerParams(use_tc_tiling_on_sc=False),
        scratch_types=[
            pltpu.VMEM((3, S, B), jnp.int32),
            pltpu.VMEM((3, S, B), jnp.float32),
            pltpu.VMEM((3, 1, B), jnp.int32),
            pltpu.VMEM((2, S, B, out_ch), jnp.float32),
            pltpu.VMEM((2, B, out_ch), jnp.float32),
            pltpu.VMEM_SHARED((NSH, out_ch), jnp.float32),
            pltpu.SemaphoreType.DMA,
            pltpu.SemaphoreType.DMA,
            pltpu.SemaphoreType.DMA,
        ],
    )
    def sc_fn(xk, basT, flatT, dste, out, idx_v, bas_v, dst_v, rows_v,
              msg_v, agg, sem_m, sem_g, sem_s):
        c = lax.axis_index("c")
        sid = lax.axis_index("s")
        wid = c * NS + sid

        z16 = jnp.zeros((16,), jnp.float32)

        def zrow(r, carry):
            for j in range(F):
                msg_v[0, r, pl.ds(j * 16, 16)] = z16
            return carry

        lax.fori_loop(0, B, zrow, 0)
        for i in range(NSH // NS // B):
            pltpu.sync_copy(msg_v.at[0],
                            agg.at[pl.ds(sid * (NSH // NS) + i * B, B)])
        plsc.subcore_barrier()

        def fire_meta(ci):
            m = lax.rem(ci, 3)
            base = wid * EW + ci * B
            pltpu.async_copy(flatT.at[:, pl.ds(base, B)], idx_v.at[m], sem_m)
            pltpu.async_copy(basT.at[:, pl.ds(base, B)], bas_v.at[m], sem_m)
            pltpu.async_copy(dste.at[pl.ds(base, B)], dst_v.at[m, 0], sem_m)

        def wait_meta():
            pltpu.make_async_copy(flatT.at[:, pl.ds(0, B)], idx_v.at[0],
                                  sem_m).wait()
            pltpu.make_async_copy(basT.at[:, pl.ds(0, B)], bas_v.at[0],
                                  sem_m).wait()
            pltpu.make_async_copy(dste.at[pl.ds(0, B)], dst_v.at[0, 0],
                                  sem_m).wait()

        def fire_gathers(ci):
            m = lax.rem(ci, 3)
            p = lax.rem(ci, 2)
            for si in range(S):
                pltpu.async_copy(xk.at[idx_v.at[m, si]], rows_v.at[p, si],
                                 sem_g)

        def wait_gathers():
            for si in range(S):
                pltpu.make_async_copy(xk.at[idx_v.at[0, si]],
                                      rows_v.at[0, si], sem_g).wait()

        fire_meta(jnp.int32(0))
        fire_meta(jnp.int32(1))
        wait_meta()
        fire_gathers(jnp.int32(0))

        def wait_scatter():
            pltpu.make_async_copy(msg_v.at[0], agg.at[dst_v.at[0, 0]],
                                  sem_s).wait()

        def chunk(ci, carry):
            m = lax.rem(ci, 3)
            p = lax.rem(ci, 2)
            wait_gathers()

            @pl.when(ci + 1 < CH)
            def _():
                wait_meta()
                fire_gathers(ci + 1)

            @pl.when(ci >= 1)
            def _():
                wait_scatter()

            @pl.when(ci + 2 < CH)
            def _():
                fire_meta(ci + 2)

            @plsc.parallel_loop(0, B // 16)
            def group(g):
                gbase = pl.multiple_of(g * 16, 16)
                bvecs = [bas_v[m, si, pl.ds(gbase, 16)] for si in range(S)]
                for i in range(16):
                    e = gbase + i
                    bb = [bvecs[si][i] for si in range(S)]
                    for j in range(F):
                        acc0 = (bb[0] * rows_v[p, 0, e, pl.ds(j * 16, 16)]
                                + bb[1] * rows_v[p, 1, e, pl.ds(j * 16, 16)])
                        acc1 = (bb[2] * rows_v[p, 2, e, pl.ds(j * 16, 16)]
                                + bb[3] * rows_v[p, 3, e, pl.ds(j * 16, 16)])
                        msg_v[p, e, pl.ds(j * 16, 16)] = acc0 + acc1
            pltpu.async_copy(msg_v.at[p], agg.at[dst_v.at[m, 0]], sem_s,
                             add=True)
            return carry

        lax.fori_loop(0, CH, chunk, 0)
        wait_scatter()
        plsc.subcore_barrier()
        rt = NSH // NS
        pltpu.sync_copy(agg.at[pl.ds(sid * rt, rt)],
                        out.at[c, pl.ds(sid * rt, rt)])

    return sc_fn


_sc32 = _make_sc(32)
_sc64 = _make_sc(64)


def kernel(x, edge_index, edge_attr, W1, root1, b1, W2, root2, b2,
           W3, root3, b3, W4, root4, b4, lw1, lb1, lw2, lb2):
    ei = edge_index.astype(jnp.int32)
    pad = EPAD - E
    p0 = jnp.pad(edge_attr[:, 0], (0, pad)).reshape(160, 1024)
    p1 = jnp.pad(edge_attr[:, 1], (0, pad)).reshape(160, 1024)
    srcp = jnp.pad(ei[0], (0, pad)).reshape(160, 1024)
    dstp = jnp.pad(ei[1], (0, pad)).reshape(160, 1024)

    bas4, flat4, dst4 = _prep(p0, p1, srcp, dstp)
    basT = bas4.reshape(S, EPAD)
    flatT = flat4.reshape(S, EPAD)
    dste = dst4.reshape(EPAD)

    def layer(h, W, root, bias, in_ch, out_ch, sc_fn):
        wmat = W.transpose(1, 0, 2).reshape(in_ch, K * out_ch)
        xk = _transform(h, wmat, in_ch, K * out_ch).reshape(N * K, out_ch)
        aggp = sc_fn(xk, basT, flatT, dste)
        return _combine(aggp, h, root, bias, in_ch, out_ch)

    h = layer(x, W1, root1, b1, 1, 32, _sc32)
    h = layer(h, W2, root2, b2, 32, 64, _sc64)
    h = layer(h, W3, root3, b3, 64, 64, _sc64)
    h = layer(h, W4, root4, b4, 64, 64, _sc64)
    return _head(h, lw1, lb1, lw2, lb2)
